# baseline (speedup 1.0000x reference)
import functools

import jax
import jax.numpy as jnp
from jax import lax
from jax.experimental import pallas as pl
from jax.experimental.pallas import tpu as pltpu
from jax.experimental.pallas import tpu_sc as plsc

_VOCAB = 100000
_DIM = 64
_EMBED_P = 0.1
_NC, _NS = 2, 16
_NW = _NC * _NS
_CH = 128


_HB = 1000


def _premask_body(s1_ref, s2_ref, w1_ref, w2_ref, o_ref):
    lo = w1_ref[...] * jnp.broadcast_to(s1_ref[...], (_HB, _DIM))
    hi = w2_ref[...] * jnp.broadcast_to(s2_ref[...], (_HB, _DIM))
    o_ref[...] = jnp.concatenate([lo, hi], axis=1)


def _premask(weight, scale):
    half = _VOCAB // 2
    grid = (half // _HB,)
    out = pl.pallas_call(
        _premask_body,
        grid=grid,
        in_specs=[
            pl.BlockSpec((_HB, 1), lambda i: (2 * i, 0)),
            pl.BlockSpec((_HB, 1), lambda i: (2 * i + 1, 0)),
            pl.BlockSpec((_HB, _DIM), lambda i: (2 * i, 0)),
            pl.BlockSpec((_HB, _DIM), lambda i: (2 * i + 1, 0)),
        ],
        out_specs=pl.BlockSpec((_HB, 2 * _DIM), lambda i: (i, 0)),
        out_shape=jax.ShapeDtypeStruct((half, 2 * _DIM), jnp.float32),
    )(scale, scale, weight, weight)
    return out.reshape(_VOCAB, _DIM)


def _sc_gather(table, idx2d):
    n_ch_total = idx2d.shape[0]
    n_ch = n_ch_total // _NW
    n_rows = n_ch_total * _CH
    mesh = plsc.VectorSubcoreMesh(core_axis_name="c", subcore_axis_name="s")

    @functools.partial(
        pl.kernel,
        out_type=jax.ShapeDtypeStruct((n_rows, _DIM), jnp.float32),
        mesh=mesh,
        scratch_types=[
            pltpu.VMEM((n_ch, _CH), jnp.int32),
            pltpu.VMEM((2, _CH, _DIM), jnp.float32),
            pltpu.SemaphoreType.DMA,
            pltpu.SemaphoreType.DMA,
            pltpu.SemaphoreType.DMA,
            pltpu.SemaphoreType.DMA,
        ],
        compiler_params=pltpu.CompilerParams(use_tc_tiling_on_sc=False),
    )
    def k(table_hbm, idx_hbm, out_hbm, idx_v, buf_v, g0, g1, w0, w1):
        wid = lax.axis_index("s") * _NC + lax.axis_index("c")
        chbase = wid * n_ch
        rowbase = chbase * _CH
        pltpu.sync_copy(idx_hbm.at[pl.ds(chbase, n_ch)], idx_v)

        def gather(j, buf, sem):
            return pltpu.make_async_copy(table_hbm.at[idx_v.at[j]], buf, sem)

        def write(j, buf, sem):
            return pltpu.make_async_copy(
                buf, out_hbm.at[pl.ds(rowbase + j * _CH, _CH)], sem)

        gather(0, buf_v.at[0], g0).start()
        gather(1, buf_v.at[1], g1).start()

        def body(s, carry):
            j0 = 2 * s
            j1 = j0 + 1
            gather(j0, buf_v.at[0], g0).wait()
            write(j0, buf_v.at[0], w0).start()
            gather(j1, buf_v.at[1], g1).wait()
            write(j1, buf_v.at[1], w1).start()

            @pl.when(j0 + 2 < n_ch)
            def _():
                write(j0, buf_v.at[0], w0).wait()
                gather(j0 + 2, buf_v.at[0], g0).start()
                write(j1, buf_v.at[1], w1).wait()
                gather(j1 + 2, buf_v.at[1], g1).start()

            return carry

        lax.fori_loop(0, n_ch // 2, body, 0)
        write(n_ch - 2, buf_v.at[0], w0).wait()
        write(n_ch - 1, buf_v.at[1], w1).wait()

    return k(table, idx2d)


def kernel(words, weight):
    keep = jax.random.bernoulli(
        jax.random.key(42), 1.0 - _EMBED_P, (_VOCAB, 1)).astype(jnp.float32)
    scale = keep / (1.0 - _EMBED_P)
    masked = _premask(weight, scale)
    flat = words.reshape(-1).astype(jnp.int32)
    chunk = flat // (2 * _HB)
    off = flat % (2 * _HB)
    pidx = chunk * (2 * _HB) + (off % _HB) * 2 + off // _HB
    idx2d = pidx.reshape(-1, _CH)
    out = _sc_gather(masked, idx2d)
    return out.reshape(words.shape + (_DIM,))


# speedup vs baseline: 1.3379x; 1.3379x over previous
import functools

import jax
import jax.numpy as jnp
from jax import lax
from jax.experimental import pallas as pl
from jax.experimental.pallas import tpu as pltpu
from jax.experimental.pallas import tpu_sc as plsc

_VOCAB = 100000
_DIM = 64
_EMBED_P = 0.1
_NC, _NS = 2, 16
_NW = _NC * _NS
_CH = 128


_RB = 2000


def _premask_body(s_ref, wt_ref, o_ref):
    wts = wt_ref[...] * jnp.broadcast_to(s_ref[0], (_DIM, _RB))
    eye = jnp.eye(_DIM, dtype=jnp.float32)
    t = jax.lax.dot_general(
        wts, eye, (((0,), (0,)), ((), ())),
        precision=jax.lax.Precision.HIGHEST)
    h = _RB // 2
    o_ref[...] = jnp.concatenate([t[:h, :], t[h:, :]], axis=1)


def _premask(weight_t, scale_row):
    half = _VOCAB // 2
    grid = (_VOCAB // _RB,)
    out = pl.pallas_call(
        _premask_body,
        grid=grid,
        in_specs=[
            pl.BlockSpec((1, _RB), lambda i: (0, i)),
            pl.BlockSpec((_DIM, _RB), lambda i: (0, i)),
        ],
        out_specs=pl.BlockSpec((_RB // 2, 2 * _DIM), lambda i: (i, 0)),
        out_shape=jax.ShapeDtypeStruct((half, 2 * _DIM), jnp.float32),
    )(scale_row, weight_t)
    return out.reshape(_VOCAB, _DIM)


def _sc_gather(table, idx2d):
    n_ch_total = idx2d.shape[0]
    n_ch = n_ch_total // _NW
    n_rows = n_ch_total * _CH
    mesh = plsc.VectorSubcoreMesh(core_axis_name="c", subcore_axis_name="s")

    @functools.partial(
        pl.kernel,
        out_type=jax.ShapeDtypeStruct((n_rows, _DIM), jnp.float32),
        mesh=mesh,
        scratch_types=[
            pltpu.VMEM((n_ch, _CH), jnp.int32),
            pltpu.VMEM((2, _CH, _DIM), jnp.float32),
            pltpu.SemaphoreType.DMA,
            pltpu.SemaphoreType.DMA,
            pltpu.SemaphoreType.DMA,
            pltpu.SemaphoreType.DMA,
        ],
        compiler_params=pltpu.CompilerParams(use_tc_tiling_on_sc=False),
    )
    def k(table_hbm, idx_hbm, out_hbm, idx_v, buf_v, g0, g1, w0, w1):
        wid = lax.axis_index("s") * _NC + lax.axis_index("c")
        chbase = wid * n_ch
        rowbase = chbase * _CH
        pltpu.sync_copy(idx_hbm.at[pl.ds(chbase, n_ch)], idx_v)

        def gather(j, buf, sem):
            return pltpu.make_async_copy(table_hbm.at[idx_v.at[j]], buf, sem)

        def write(j, buf, sem):
            return pltpu.make_async_copy(
                buf, out_hbm.at[pl.ds(rowbase + j * _CH, _CH)], sem)

        gather(0, buf_v.at[0], g0).start()
        gather(1, buf_v.at[1], g1).start()

        def body(s, carry):
            j0 = 2 * s
            j1 = j0 + 1
            gather(j0, buf_v.at[0], g0).wait()
            write(j0, buf_v.at[0], w0).start()
            gather(j1, buf_v.at[1], g1).wait()
            write(j1, buf_v.at[1], w1).start()

            @pl.when(j0 + 2 < n_ch)
            def _():
                write(j0, buf_v.at[0], w0).wait()
                gather(j0 + 2, buf_v.at[0], g0).start()
                write(j1, buf_v.at[1], w1).wait()
                gather(j1 + 2, buf_v.at[1], g1).start()

            return carry

        lax.fori_loop(0, n_ch // 2, body, 0)
        write(n_ch - 2, buf_v.at[0], w0).wait()
        write(n_ch - 1, buf_v.at[1], w1).wait()

    return k(table, idx2d)


def kernel(words, weight):
    keep = jax.random.bernoulli(
        jax.random.key(42), 1.0 - _EMBED_P, (_VOCAB,)).astype(jnp.float32)
    scale = jax.lax.optimization_barrier(keep / (1.0 - _EMBED_P))
    masked = _premask(jnp.transpose(weight), scale.reshape(1, _VOCAB))
    flat = words.reshape(-1).astype(jnp.int32)
    hb = _RB // 2
    off = flat % _RB
    pidx = (flat - off) + (off % hb) * 2 + off // hb
    idx2d = pidx.reshape(-1, _CH)
    out = _sc_gather(masked, idx2d)
    return out.reshape(words.shape + (_DIM,))
